# Initial kernel scaffold; baseline (speedup 1.0000x reference)
import jax
import jax.numpy as jnp
from jax.experimental import pallas as pl


def kernel(x, edge_index, W_pool, W_read, b_read):
    raise NotImplementedError("write your pallas kernel here")


# speedup vs baseline: 58.4245x; 58.4245x over previous
import functools

import jax
import jax.numpy as jnp
from jax import lax
from jax.experimental import pallas as pl
from jax.experimental.pallas import tpu as pltpu
from jax.experimental.pallas import tpu_sc as plsc

_SC_PARAMS = pltpu.CompilerParams(needs_layout_passes=False,
                                  use_tc_tiling_on_sc=False)

N_NODES = 10000
N_EDGES = 320000
NODE_LEN = 128

_info = plsc.get_sparse_core_info()
NC, NS, L = _info.num_cores, _info.num_subcores, _info.num_lanes
NW = NC * NS
E_PER_W = N_EDGES // NW
N_PAD = ((N_NODES + NW * L - 1) // (NW * L)) * (NW * L)
COLS_PER_W = N_PAD // NW


def _k1_body(x_ref, wp_ref, wr_ref, s_ref):
    v128 = jnp.dot(wr_ref[...], wp_ref[:, :NODE_LEN],
                   preferred_element_type=jnp.float32)
    s_ref[...] = jnp.dot(x_ref[...], v128.T,
                         preferred_element_type=jnp.float32)


def _k1(x, W_pool, W_read):
    blk = 1000
    grid = N_NODES // blk
    return pl.pallas_call(
        _k1_body,
        grid=(grid,),
        in_specs=[
            pl.BlockSpec((blk, NODE_LEN), lambda i: (i, 0)),
            pl.BlockSpec(W_pool.shape, lambda i: (0, 0)),
            pl.BlockSpec(W_read.shape, lambda i: (0, 0)),
        ],
        out_specs=pl.BlockSpec((blk, 1), lambda i: (i, 0)),
        out_shape=jax.ShapeDtypeStruct((N_NODES, 1), jnp.float32),
    )(x, W_pool, W_read)


def _k2_body(src_hbm, dst_hbm, s_hbm, tpart_hbm, dpart_hbm,
             src_v, dst_v, s_v, tacc_v, dacc_v):
    wid = lax.axis_index("s") * NC + lax.axis_index("c")
    base = wid * E_PER_W
    pltpu.sync_copy(src_hbm.at[pl.ds(base, E_PER_W)], src_v)
    pltpu.sync_copy(dst_hbm.at[pl.ds(base, E_PER_W)], dst_v)
    pltpu.sync_copy(s_hbm, s_v)

    zeros = jnp.zeros((L,), jnp.float32)

    def zero_body(i, _):
        tacc_v[pl.ds(i * L, L)] = zeros
        dacc_v[pl.ds(i * L, L)] = zeros
        return 0

    lax.fori_loop(0, N_PAD // L, zero_body, 0)

    ones = jnp.ones((L,), jnp.float32)

    def edge_body(i, _):
        off = i * L
        i_s = src_v[pl.ds(off, L)]
        i_d = dst_v[pl.ds(off, L)]
        vals = plsc.load_gather(s_v, [i_s])
        plsc.addupdate_scatter(tacc_v, [i_d], vals)
        plsc.addupdate_scatter(dacc_v, [i_s], ones)
        return 0

    lax.fori_loop(0, E_PER_W // L, edge_body, 0)

    pltpu.sync_copy(tacc_v, tpart_hbm.at[wid])
    pltpu.sync_copy(dacc_v, dpart_hbm.at[wid])


def _k2(src, dst, s):
    mesh = plsc.VectorSubcoreMesh(core_axis_name="c", subcore_axis_name="s")
    f = pl.kernel(
        _k2_body,
        mesh=mesh,
        compiler_params=_SC_PARAMS,
        out_type=(
            jax.ShapeDtypeStruct((NW, N_PAD), jnp.float32),
            jax.ShapeDtypeStruct((NW, N_PAD), jnp.float32),
        ),
        scratch_types=[
            pltpu.VMEM((E_PER_W,), jnp.int32),
            pltpu.VMEM((E_PER_W,), jnp.int32),
            pltpu.VMEM((N_NODES,), jnp.float32),
            pltpu.VMEM((N_PAD,), jnp.float32),
            pltpu.VMEM((N_PAD,), jnp.float32),
        ],
    )
    return f(src, dst, s)


def _k3_body(tpart_hbm, dpart_hbm, t_hbm, deg_hbm, buf_v, acc_v):
    wid = lax.axis_index("s") * NC + lax.axis_index("c")
    c0 = wid * COLS_PER_W

    for part_hbm, out_hbm in ((tpart_hbm, t_hbm), (dpart_hbm, deg_hbm)):
        for r in range(NW):
            pltpu.sync_copy(part_hbm.at[r, pl.ds(c0, COLS_PER_W)], buf_v.at[r])

        def col_body(j, _):
            acc = jnp.zeros((L,), jnp.float32)

            def row_body(r, a):
                return a + buf_v[r, pl.ds(j * L, L)]

            acc = lax.fori_loop(0, NW, row_body, acc)
            acc_v[pl.ds(j * L, L)] = acc
            return 0

        lax.fori_loop(0, COLS_PER_W // L, col_body, 0)
        pltpu.sync_copy(acc_v, out_hbm.at[pl.ds(c0, COLS_PER_W)])


def _k3(tpart, dpart):
    mesh = plsc.VectorSubcoreMesh(core_axis_name="c", subcore_axis_name="s")
    f = pl.kernel(
        _k3_body,
        mesh=mesh,
        compiler_params=_SC_PARAMS,
        out_type=(
            jax.ShapeDtypeStruct((N_PAD,), jnp.float32),
            jax.ShapeDtypeStruct((N_PAD,), jnp.float32),
        ),
        scratch_types=[
            pltpu.VMEM((NW, COLS_PER_W), jnp.float32),
            pltpu.VMEM((COLS_PER_W,), jnp.float32),
        ],
    )
    return f(tpart, dpart)


def _k4_body(src_hbm, dst_hbm, t_hbm, deg_hbm, out_hbm,
             src_v, dst_v, t_v, deg_v, res_v):
    wid = lax.axis_index("s") * NC + lax.axis_index("c")
    base = wid * E_PER_W
    pltpu.sync_copy(src_hbm.at[pl.ds(base, E_PER_W)], src_v)
    pltpu.sync_copy(dst_hbm.at[pl.ds(base, E_PER_W)], dst_v)
    pltpu.sync_copy(t_hbm.at[pl.ds(0, N_NODES)], t_v)
    pltpu.sync_copy(deg_hbm.at[pl.ds(0, N_NODES)], deg_v)

    def edge_body(i, acc):
        off = i * L
        i_s = src_v[pl.ds(off, L)]
        i_d = dst_v[pl.ds(off, L)]
        tv = plsc.load_gather(t_v, [i_s])
        dv = plsc.load_gather(deg_v, [i_d])
        return acc + tv * dv

    acc = lax.fori_loop(0, E_PER_W // L, edge_body,
                        jnp.zeros((L,), jnp.float32))
    res_v[...] = acc
    pltpu.sync_copy(res_v, out_hbm.at[wid])


def _k4(src, dst, t, deg):
    mesh = plsc.VectorSubcoreMesh(core_axis_name="c", subcore_axis_name="s")
    f = pl.kernel(
        _k4_body,
        mesh=mesh,
        compiler_params=_SC_PARAMS,
        out_type=jax.ShapeDtypeStruct((NW, L), jnp.float32),
        scratch_types=[
            pltpu.VMEM((E_PER_W,), jnp.int32),
            pltpu.VMEM((E_PER_W,), jnp.int32),
            pltpu.VMEM((N_NODES,), jnp.float32),
            pltpu.VMEM((N_NODES,), jnp.float32),
            pltpu.VMEM((L,), jnp.float32),
        ],
    )
    return f(src, dst, t, deg)


def _k5_body(parts_ref, b_ref, out_ref):
    total = jnp.sum(parts_ref[...]) * (1.0 / N_NODES) + b_ref[0, 0]
    out_ref[...] = total.reshape(1, 1)


def _k5(parts, b_read):
    return pl.pallas_call(
        _k5_body,
        out_shape=jax.ShapeDtypeStruct((1, 1), jnp.float32),
    )(parts, b_read.reshape(1, 1))


def kernel(x, edge_index, W_pool, W_read, b_read):
    src = edge_index[0].astype(jnp.int32)
    dst = edge_index[1].astype(jnp.int32)
    s = _k1(x, W_pool, W_read).reshape(N_NODES)
    tpart, dpart = _k2(src, dst, s)
    t, deg = _k3(tpart, dpart)
    parts = _k4(src, dst, t, deg)
    out = _k5(parts, b_read)
    return out.reshape(1)


# speedup vs baseline: 83.0532x; 1.4215x over previous
import jax
import jax.numpy as jnp
from jax import lax
from jax.experimental import pallas as pl
from jax.experimental.pallas import tpu as pltpu
from jax.experimental.pallas import tpu_sc as plsc

_SC_PARAMS = pltpu.CompilerParams(needs_layout_passes=False,
                                  use_tc_tiling_on_sc=False)

N_NODES = 10000
N_EDGES = 320000
NODE_LEN = 128

_info = plsc.get_sparse_core_info()
NC, NS, L = _info.num_cores, _info.num_subcores, _info.num_lanes
NW = NC * NS
E_PER_W = N_EDGES // NW
N_PAD = ((N_NODES + NW * L - 1) // (NW * L)) * (NW * L)
COLS2 = N_PAD // NS


def _k1_body(x_ref, wp_ref, wr_ref, s_ref):
    v128 = jnp.dot(wr_ref[...], wp_ref[:, :NODE_LEN],
                   preferred_element_type=jnp.float32)
    s_ref[...] = jnp.dot(x_ref[...], v128.T,
                         preferred_element_type=jnp.float32)


def _k1(x, W_pool, W_read):
    blk = 1000
    grid = N_NODES // blk
    return pl.pallas_call(
        _k1_body,
        grid=(grid,),
        in_specs=[
            pl.BlockSpec((blk, NODE_LEN), lambda i: (i, 0)),
            pl.BlockSpec(W_pool.shape, lambda i: (0, 0)),
            pl.BlockSpec(W_read.shape, lambda i: (0, 0)),
        ],
        out_specs=pl.BlockSpec((blk, 1), lambda i: (i, 0)),
        out_shape=jax.ShapeDtypeStruct((N_NODES, 1), jnp.float32),
    )(x, W_pool, W_read)


def _k2_body(src_hbm, dst_hbm, s_hbm, tpart_hbm, dpart_hbm,
             src_v, dst_v, s_v, tacc_v, dacc_v, buf_v, red_v, tsh, dsh):
    cid = lax.axis_index("c")
    sid = lax.axis_index("s")
    wid = sid * NC + cid
    base = wid * E_PER_W
    pltpu.sync_copy(src_hbm.at[pl.ds(base, E_PER_W)], src_v)
    pltpu.sync_copy(dst_hbm.at[pl.ds(base, E_PER_W)], dst_v)
    pltpu.sync_copy(s_hbm, s_v)

    zeros = jnp.zeros((L,), jnp.float32)

    @plsc.parallel_loop(0, N_PAD // L, unroll=8)
    def _(i):
        tacc_v[pl.ds(i * L, L)] = zeros
        dacc_v[pl.ds(i * L, L)] = zeros

    ones = jnp.ones((L,), jnp.float32)

    @plsc.parallel_loop(0, E_PER_W // L, unroll=8)
    def _(i):
        off = i * L
        i_s = src_v[pl.ds(off, L)]
        i_d = dst_v[pl.ds(off, L)]
        vals = plsc.load_gather(s_v, [i_s])
        plsc.addupdate_scatter(tacc_v, [i_d], vals)
        plsc.addupdate_scatter(dacc_v, [i_s], ones)

    pltpu.sync_copy(tacc_v, tsh.at[sid])
    pltpu.sync_copy(dacc_v, dsh.at[sid])
    plsc.subcore_barrier()

    c0 = sid * COLS2
    for arr_sh, out_hbm in ((tsh, tpart_hbm), (dsh, dpart_hbm)):
        for r in range(NS):
            pltpu.sync_copy(arr_sh.at[r, pl.ds(c0, COLS2)], buf_v.at[r])

        @plsc.parallel_loop(0, COLS2 // L, unroll=4)
        def _(j):
            acc = buf_v[0, pl.ds(j * L, L)]
            for r in range(1, NS):
                acc = acc + buf_v[r, pl.ds(j * L, L)]
            red_v[pl.ds(j * L, L)] = acc

        pltpu.sync_copy(red_v, out_hbm.at[cid, pl.ds(c0, COLS2)])


def _k2(src, dst, s):
    mesh = plsc.VectorSubcoreMesh(core_axis_name="c", subcore_axis_name="s")
    f = pl.kernel(
        _k2_body,
        mesh=mesh,
        compiler_params=_SC_PARAMS,
        out_type=(
            jax.ShapeDtypeStruct((NC, N_PAD), jnp.float32),
            jax.ShapeDtypeStruct((NC, N_PAD), jnp.float32),
        ),
        scratch_types=[
            pltpu.VMEM((E_PER_W,), jnp.int32),
            pltpu.VMEM((E_PER_W,), jnp.int32),
            pltpu.VMEM((N_NODES,), jnp.float32),
            pltpu.VMEM((N_PAD,), jnp.float32),
            pltpu.VMEM((N_PAD,), jnp.float32),
            pltpu.VMEM((NS, COLS2), jnp.float32),
            pltpu.VMEM((COLS2,), jnp.float32),
            pltpu.VMEM_SHARED((NS, N_PAD), jnp.float32),
            pltpu.VMEM_SHARED((NS, N_PAD), jnp.float32),
        ],
    )
    return f(src, dst, s)


def _k4_body(src_hbm, dst_hbm, t_hbm, deg_hbm, out_hbm,
             src_v, dst_v, t_v, deg_v, tmp_v, res_v):
    wid = lax.axis_index("s") * NC + lax.axis_index("c")
    base = wid * E_PER_W
    pltpu.sync_copy(src_hbm.at[pl.ds(base, E_PER_W)], src_v)
    pltpu.sync_copy(dst_hbm.at[pl.ds(base, E_PER_W)], dst_v)
    pltpu.sync_copy(t_hbm.at[0, pl.ds(0, N_NODES)], t_v)
    pltpu.sync_copy(t_hbm.at[1, pl.ds(0, N_NODES)], tmp_v)

    @plsc.parallel_loop(0, N_NODES // L, unroll=8)
    def _(i):
        sl = pl.ds(i * L, L)
        t_v[sl] = t_v[sl] + tmp_v[sl]

    pltpu.sync_copy(deg_hbm.at[0, pl.ds(0, N_NODES)], deg_v)
    pltpu.sync_copy(deg_hbm.at[1, pl.ds(0, N_NODES)], tmp_v)

    @plsc.parallel_loop(0, N_NODES // L, unroll=8)
    def _(i):
        sl = pl.ds(i * L, L)
        deg_v[sl] = deg_v[sl] + tmp_v[sl]

    @plsc.parallel_loop(0, E_PER_W // L, unroll=8,
                        carry=jnp.zeros((L,), jnp.float32))
    def acc(i, a):
        off = i * L
        i_s = src_v[pl.ds(off, L)]
        i_d = dst_v[pl.ds(off, L)]
        tv = plsc.load_gather(t_v, [i_s])
        dv = plsc.load_gather(deg_v, [i_d])
        return a + tv * dv

    res_v[...] = acc
    pltpu.sync_copy(res_v, out_hbm.at[wid])


def _k4(src, dst, t2, deg2):
    mesh = plsc.VectorSubcoreMesh(core_axis_name="c", subcore_axis_name="s")
    f = pl.kernel(
        _k4_body,
        mesh=mesh,
        compiler_params=_SC_PARAMS,
        out_type=jax.ShapeDtypeStruct((NW, L), jnp.float32),
        scratch_types=[
            pltpu.VMEM((E_PER_W,), jnp.int32),
            pltpu.VMEM((E_PER_W,), jnp.int32),
            pltpu.VMEM((N_NODES,), jnp.float32),
            pltpu.VMEM((N_NODES,), jnp.float32),
            pltpu.VMEM((N_NODES,), jnp.float32),
            pltpu.VMEM((L,), jnp.float32),
        ],
    )
    return f(src, dst, t2, deg2)


def _k5_body(parts_ref, b_ref, out_ref):
    total = jnp.sum(parts_ref[...]) * (1.0 / N_NODES) + b_ref[0, 0]
    out_ref[...] = total.reshape(1, 1)


def _k5(parts, b_read):
    return pl.pallas_call(
        _k5_body,
        out_shape=jax.ShapeDtypeStruct((1, 1), jnp.float32),
    )(parts, b_read.reshape(1, 1))


def kernel(x, edge_index, W_pool, W_read, b_read):
    src = edge_index[0].astype(jnp.int32)
    dst = edge_index[1].astype(jnp.int32)
    s = _k1(x, W_pool, W_read).reshape(N_NODES)
    tpart, dpart = _k2(src, dst, s)
    parts = _k4(src, dst, tpart, dpart)
    out = _k5(parts, b_read)
    return out.reshape(1)


# speedup vs baseline: 108.8162x; 1.3102x over previous
import jax
import jax.numpy as jnp
from jax import lax
from jax.experimental import pallas as pl
from jax.experimental.pallas import tpu as pltpu
from jax.experimental.pallas import tpu_sc as plsc

_SC_PARAMS = pltpu.CompilerParams(needs_layout_passes=False,
                                  use_tc_tiling_on_sc=False)

N_NODES = 10000
N_EDGES = 320000
NODE_LEN = 128

_info = plsc.get_sparse_core_info()
NC, NS, L = _info.num_cores, _info.num_subcores, _info.num_lanes
NW = NC * NS
E_PER_W = N_EDGES // NW
N_PAD = ((N_NODES + NW * L - 1) // (NW * L)) * (NW * L)
COLS2 = N_PAD // NS


def _k1_body(x_ref, wp_ref, wr_ref, s_ref):
    v128 = jnp.dot(wr_ref[...], wp_ref[:, :NODE_LEN],
                   preferred_element_type=jnp.float32)
    sblk = jnp.dot(x_ref[...], v128.T,
                   preferred_element_type=jnp.float32)
    s_ref[...] = sblk.T


def _k1(x, W_pool, W_read):
    blk = 2048
    grid = N_PAD // blk
    return pl.pallas_call(
        _k1_body,
        grid=(grid,),
        in_specs=[
            pl.BlockSpec((blk, NODE_LEN), lambda i: (i, 0)),
            pl.BlockSpec(W_pool.shape, lambda i: (0, 0)),
            pl.BlockSpec(W_read.shape, lambda i: (0, 0)),
        ],
        out_specs=pl.BlockSpec((1, blk), lambda i: (0, i)),
        out_shape=jax.ShapeDtypeStruct((1, N_PAD), jnp.float32),
    )(x, W_pool, W_read)


def _k2_body(ei_hbm, s_hbm, tpart_hbm, dpart_hbm,
             src_v, dst_v, s_v, tacc_v, dacc_v, buf_v, red_v, tsh, dsh):
    cid = lax.axis_index("c")
    sid = lax.axis_index("s")
    wid = sid * NC + cid
    base = wid * E_PER_W
    pltpu.sync_copy(ei_hbm.at[0, pl.ds(base, E_PER_W)], src_v)
    pltpu.sync_copy(ei_hbm.at[1, pl.ds(base, E_PER_W)], dst_v)
    pltpu.sync_copy(s_hbm.at[0], s_v)

    zeros = jnp.zeros((L,), jnp.float32)

    @plsc.parallel_loop(0, N_PAD // L, unroll=8)
    def _(i):
        tacc_v[pl.ds(i * L, L)] = zeros
        dacc_v[pl.ds(i * L, L)] = zeros

    ones = jnp.ones((L,), jnp.float32)

    @plsc.parallel_loop(0, E_PER_W // L, unroll=8)
    def _(i):
        off = i * L
        i_s = src_v[pl.ds(off, L)]
        i_d = dst_v[pl.ds(off, L)]
        vals = plsc.load_gather(s_v, [i_s])
        plsc.addupdate_scatter(tacc_v, [i_d], vals)
        plsc.addupdate_scatter(dacc_v, [i_s], ones)

    pltpu.sync_copy(tacc_v, tsh.at[sid])
    pltpu.sync_copy(dacc_v, dsh.at[sid])
    plsc.subcore_barrier()

    c0 = sid * COLS2
    for arr_sh, out_hbm in ((tsh, tpart_hbm), (dsh, dpart_hbm)):
        for r in range(NS):
            pltpu.sync_copy(arr_sh.at[r, pl.ds(c0, COLS2)], buf_v.at[r])

        @plsc.parallel_loop(0, COLS2 // L, unroll=4)
        def _(j):
            acc = buf_v[0, pl.ds(j * L, L)]
            for r in range(1, NS):
                acc = acc + buf_v[r, pl.ds(j * L, L)]
            red_v[pl.ds(j * L, L)] = acc

        pltpu.sync_copy(red_v, out_hbm.at[cid, pl.ds(c0, COLS2)])


def _k2(ei, s):
    mesh = plsc.VectorSubcoreMesh(core_axis_name="c", subcore_axis_name="s")
    f = pl.kernel(
        _k2_body,
        mesh=mesh,
        compiler_params=_SC_PARAMS,
        out_type=(
            jax.ShapeDtypeStruct((NC, N_PAD), jnp.float32),
            jax.ShapeDtypeStruct((NC, N_PAD), jnp.float32),
        ),
        scratch_types=[
            pltpu.VMEM((E_PER_W,), jnp.int32),
            pltpu.VMEM((E_PER_W,), jnp.int32),
            pltpu.VMEM((N_PAD,), jnp.float32),
            pltpu.VMEM((N_PAD,), jnp.float32),
            pltpu.VMEM((N_PAD,), jnp.float32),
            pltpu.VMEM((NS, COLS2), jnp.float32),
            pltpu.VMEM((COLS2,), jnp.float32),
            pltpu.VMEM_SHARED((NS, N_PAD), jnp.float32),
            pltpu.VMEM_SHARED((NS, N_PAD), jnp.float32),
        ],
    )
    return f(ei, s)


def _k4_body(ei_hbm, t_hbm, deg_hbm, out_hbm,
             src_v, dst_v, t_v, deg_v, tmp_v, res_v):
    wid = lax.axis_index("s") * NC + lax.axis_index("c")
    base = wid * E_PER_W
    pltpu.sync_copy(ei_hbm.at[0, pl.ds(base, E_PER_W)], src_v)
    pltpu.sync_copy(ei_hbm.at[1, pl.ds(base, E_PER_W)], dst_v)
    pltpu.sync_copy(t_hbm.at[0, pl.ds(0, N_NODES)], t_v)
    pltpu.sync_copy(t_hbm.at[1, pl.ds(0, N_NODES)], tmp_v)

    @plsc.parallel_loop(0, N_NODES // L, unroll=8)
    def _(i):
        sl = pl.ds(i * L, L)
        t_v[sl] = t_v[sl] + tmp_v[sl]

    pltpu.sync_copy(deg_hbm.at[0, pl.ds(0, N_NODES)], deg_v)
    pltpu.sync_copy(deg_hbm.at[1, pl.ds(0, N_NODES)], tmp_v)

    @plsc.parallel_loop(0, N_NODES // L, unroll=8)
    def _(i):
        sl = pl.ds(i * L, L)
        deg_v[sl] = deg_v[sl] + tmp_v[sl]

    @plsc.parallel_loop(0, E_PER_W // L, unroll=8,
                        carry=jnp.zeros((L,), jnp.float32))
    def acc(i, a):
        off = i * L
        i_s = src_v[pl.ds(off, L)]
        i_d = dst_v[pl.ds(off, L)]
        tv = plsc.load_gather(t_v, [i_s])
        dv = plsc.load_gather(deg_v, [i_d])
        return a + tv * dv

    res_v[...] = acc
    pltpu.sync_copy(res_v, out_hbm.at[wid])


def _k4(ei, t2, deg2):
    mesh = plsc.VectorSubcoreMesh(core_axis_name="c", subcore_axis_name="s")
    f = pl.kernel(
        _k4_body,
        mesh=mesh,
        compiler_params=_SC_PARAMS,
        out_type=jax.ShapeDtypeStruct((NW, L), jnp.float32),
        scratch_types=[
            pltpu.VMEM((E_PER_W,), jnp.int32),
            pltpu.VMEM((E_PER_W,), jnp.int32),
            pltpu.VMEM((N_NODES,), jnp.float32),
            pltpu.VMEM((N_NODES,), jnp.float32),
            pltpu.VMEM((N_NODES,), jnp.float32),
            pltpu.VMEM((L,), jnp.float32),
        ],
    )
    return f(ei, t2, deg2)


def _k5_body(parts_ref, b_ref, out_ref):
    total = jnp.sum(parts_ref[...]) * (1.0 / N_NODES) + b_ref[0, 0]
    out_ref[...] = total.reshape(1, 1)


def _k5(parts, b_read):
    return pl.pallas_call(
        _k5_body,
        out_shape=jax.ShapeDtypeStruct((1, 1), jnp.float32),
    )(parts, b_read.reshape(1, 1))


def kernel(x, edge_index, W_pool, W_read, b_read):
    ei = edge_index.astype(jnp.int32)
    s = _k1(x, W_pool, W_read)
    tpart, dpart = _k2(ei, s)
    parts = _k4(ei, tpart, dpart)
    out = _k5(parts, b_read)
    return out.reshape(1)


# speedup vs baseline: 121.6047x; 1.1175x over previous
import jax
import jax.numpy as jnp
from jax import lax
from jax.experimental import pallas as pl
from jax.experimental.pallas import tpu as pltpu
from jax.experimental.pallas import tpu_sc as plsc

_SC_PARAMS = pltpu.CompilerParams(needs_layout_passes=False,
                                  use_tc_tiling_on_sc=False)

N_NODES = 10000
N_EDGES = 320000
NODE_LEN = 128

_info = plsc.get_sparse_core_info()
NC, NS, L = _info.num_cores, _info.num_subcores, _info.num_lanes
NW = NC * NS
E_PER_W = N_EDGES // NW
N_PAD = ((N_NODES + NW * L - 1) // (NW * L)) * (NW * L)
COLS2 = N_PAD // NS


def _k1_body(x_ref, wp_ref, wr_ref, s_ref):
    v128 = jnp.dot(wr_ref[...], wp_ref[:, :NODE_LEN],
                   preferred_element_type=jnp.float32)
    sblk = jnp.dot(x_ref[...], v128.T,
                   preferred_element_type=jnp.float32)
    s_ref[...] = sblk.T


def _k1(x, W_pool, W_read):
    blk = 2048
    grid = N_PAD // blk
    return pl.pallas_call(
        _k1_body,
        grid=(grid,),
        in_specs=[
            pl.BlockSpec((blk, NODE_LEN), lambda i: (i, 0)),
            pl.BlockSpec(W_pool.shape, lambda i: (0, 0)),
            pl.BlockSpec(W_read.shape, lambda i: (0, 0)),
        ],
        out_specs=pl.BlockSpec((1, blk), lambda i: (0, i)),
        out_shape=jax.ShapeDtypeStruct((1, N_PAD), jnp.float32),
    )(x, W_pool, W_read)


def _k2_body(ei_hbm, s_hbm, tpart_hbm, dpart_hbm,
             src_v, dst_v, s_v, tacc_v, dacc_v, buf_v, red_v, tsh, dsh, sem):
    cid = lax.axis_index("c")
    sid = lax.axis_index("s")
    wid = sid * NC + cid
    base = wid * E_PER_W
    c1 = pltpu.async_copy(ei_hbm.at[0, pl.ds(base, E_PER_W)], src_v, sem)
    c2 = pltpu.async_copy(ei_hbm.at[1, pl.ds(base, E_PER_W)], dst_v, sem)
    c3 = pltpu.async_copy(s_hbm.at[0], s_v, sem)
    c1.wait()
    c2.wait()
    c3.wait()

    zeros = jnp.zeros((L,), jnp.float32)

    @plsc.parallel_loop(0, N_PAD // L, unroll=8)
    def _(i):
        tacc_v[pl.ds(i * L, L)] = zeros
        dacc_v[pl.ds(i * L, L)] = zeros

    ones = jnp.ones((L,), jnp.float32)

    @plsc.parallel_loop(0, E_PER_W // L, unroll=8)
    def _(i):
        off = i * L
        i_s = src_v[pl.ds(off, L)]
        i_d = dst_v[pl.ds(off, L)]
        vals = plsc.load_gather(s_v, [i_s])
        plsc.addupdate_scatter(tacc_v, [i_d], vals)
        plsc.addupdate_scatter(dacc_v, [i_s], ones)

    p1 = pltpu.async_copy(tacc_v, tsh.at[sid], sem)
    p2 = pltpu.async_copy(dacc_v, dsh.at[sid], sem)
    p1.wait()
    p2.wait()
    plsc.subcore_barrier()

    c0 = sid * COLS2
    for arr_sh, out_hbm in ((tsh, tpart_hbm), (dsh, dpart_hbm)):
        cps = [pltpu.async_copy(arr_sh.at[r, pl.ds(c0, COLS2)], buf_v.at[r], sem)
               for r in range(NS)]
        for cp in cps:
            cp.wait()

        @plsc.parallel_loop(0, COLS2 // L, unroll=4)
        def _(j):
            acc = buf_v[0, pl.ds(j * L, L)]
            for r in range(1, NS):
                acc = acc + buf_v[r, pl.ds(j * L, L)]
            red_v[pl.ds(j * L, L)] = acc

        pltpu.sync_copy(red_v, out_hbm.at[cid, pl.ds(c0, COLS2)])


def _k2(ei, s):
    mesh = plsc.VectorSubcoreMesh(core_axis_name="c", subcore_axis_name="s")
    f = pl.kernel(
        _k2_body,
        mesh=mesh,
        compiler_params=_SC_PARAMS,
        out_type=(
            jax.ShapeDtypeStruct((NC, N_PAD), jnp.float32),
            jax.ShapeDtypeStruct((NC, N_PAD), jnp.float32),
        ),
        scratch_types=[
            pltpu.VMEM((E_PER_W,), jnp.int32),
            pltpu.VMEM((E_PER_W,), jnp.int32),
            pltpu.VMEM((N_PAD,), jnp.float32),
            pltpu.VMEM((N_PAD,), jnp.float32),
            pltpu.VMEM((N_PAD,), jnp.float32),
            pltpu.VMEM((NS, COLS2), jnp.float32),
            pltpu.VMEM((COLS2,), jnp.float32),
            pltpu.VMEM_SHARED((NS, N_PAD), jnp.float32),
            pltpu.VMEM_SHARED((NS, N_PAD), jnp.float32),
            pltpu.SemaphoreType.DMA,
        ],
    )
    return f(ei, s)


def _k4_body(ei_hbm, t_hbm, deg_hbm, out_hbm,
             src_v, dst_v, t_v, deg_v, tmp_v, tmp2_v, res_v, sem):
    wid = lax.axis_index("s") * NC + lax.axis_index("c")
    base = wid * E_PER_W
    cps = [
        pltpu.async_copy(ei_hbm.at[0, pl.ds(base, E_PER_W)], src_v, sem),
        pltpu.async_copy(ei_hbm.at[1, pl.ds(base, E_PER_W)], dst_v, sem),
        pltpu.async_copy(t_hbm.at[0, pl.ds(0, N_NODES)], t_v, sem),
        pltpu.async_copy(t_hbm.at[1, pl.ds(0, N_NODES)], tmp_v, sem),
        pltpu.async_copy(deg_hbm.at[0, pl.ds(0, N_NODES)], deg_v, sem),
        pltpu.async_copy(deg_hbm.at[1, pl.ds(0, N_NODES)], tmp2_v, sem),
    ]
    for cp in cps:
        cp.wait()

    @plsc.parallel_loop(0, N_NODES // L, unroll=8)
    def _(i):
        sl = pl.ds(i * L, L)
        t_v[sl] = t_v[sl] + tmp_v[sl]
        deg_v[sl] = deg_v[sl] + tmp2_v[sl]

    @plsc.parallel_loop(0, E_PER_W // L, unroll=8,
                        carry=jnp.zeros((L,), jnp.float32))
    def acc(i, a):
        off = i * L
        i_s = src_v[pl.ds(off, L)]
        i_d = dst_v[pl.ds(off, L)]
        tv = plsc.load_gather(t_v, [i_s])
        dv = plsc.load_gather(deg_v, [i_d])
        return a + tv * dv

    res_v[...] = acc
    pltpu.sync_copy(res_v, out_hbm.at[wid])


def _k4(ei, t2, deg2):
    mesh = plsc.VectorSubcoreMesh(core_axis_name="c", subcore_axis_name="s")
    f = pl.kernel(
        _k4_body,
        mesh=mesh,
        compiler_params=_SC_PARAMS,
        out_type=jax.ShapeDtypeStruct((NW, L), jnp.float32),
        scratch_types=[
            pltpu.VMEM((E_PER_W,), jnp.int32),
            pltpu.VMEM((E_PER_W,), jnp.int32),
            pltpu.VMEM((N_NODES,), jnp.float32),
            pltpu.VMEM((N_NODES,), jnp.float32),
            pltpu.VMEM((N_NODES,), jnp.float32),
            pltpu.VMEM((N_NODES,), jnp.float32),
            pltpu.VMEM((L,), jnp.float32),
            pltpu.SemaphoreType.DMA,
        ],
    )
    return f(ei, t2, deg2)


def _k5_body(parts_ref, b_ref, out_ref):
    total = jnp.sum(parts_ref[...]) * (1.0 / N_NODES) + b_ref[0, 0]
    out_ref[...] = total.reshape(1, 1)


def _k5(parts, b_read):
    return pl.pallas_call(
        _k5_body,
        out_shape=jax.ShapeDtypeStruct((1, 1), jnp.float32),
    )(parts, b_read.reshape(1, 1))


def kernel(x, edge_index, W_pool, W_read, b_read):
    ei = edge_index.astype(jnp.int32)
    s = _k1(x, W_pool, W_read)
    tpart, dpart = _k2(ei, s)
    parts = _k4(ei, tpart, dpart)
    out = _k5(parts, b_read)
    return out.reshape(1)


# speedup vs baseline: 122.7095x; 1.0091x over previous
import jax
import jax.numpy as jnp
from jax import lax
from jax.experimental import pallas as pl
from jax.experimental.pallas import tpu as pltpu
from jax.experimental.pallas import tpu_sc as plsc

_SC_PARAMS = pltpu.CompilerParams(needs_layout_passes=False,
                                  use_tc_tiling_on_sc=False)

N_NODES = 10000
N_EDGES = 320000
NODE_LEN = 128

_info = plsc.get_sparse_core_info()
NC, NS, L = _info.num_cores, _info.num_subcores, _info.num_lanes
NW = NC * NS
E_PER_W = N_EDGES // NW
N_PAD = ((N_NODES + NW * L - 1) // (NW * L)) * (NW * L)
COLS2 = N_PAD // NS


def _k1_body(x_ref, wp_ref, wr_ref, s_ref):
    v128 = jnp.dot(wr_ref[...], wp_ref[:, :NODE_LEN],
                   preferred_element_type=jnp.float32)
    s_ref[...] = lax.dot_general(
        v128, x_ref[...], (((1,), (1,)), ((), ())),
        preferred_element_type=jnp.float32)


def _k1(x, W_pool, W_read):
    blk = 2048
    grid = N_PAD // blk
    return pl.pallas_call(
        _k1_body,
        grid=(grid,),
        in_specs=[
            pl.BlockSpec((blk, NODE_LEN), lambda i: (i, 0)),
            pl.BlockSpec(W_pool.shape, lambda i: (0, 0)),
            pl.BlockSpec(W_read.shape, lambda i: (0, 0)),
        ],
        out_specs=pl.BlockSpec((1, blk), lambda i: (0, i)),
        out_shape=jax.ShapeDtypeStruct((1, N_PAD), jnp.float32),
    )(x, W_pool, W_read)


def _k2_body(ei_hbm, s_hbm, tpart_hbm, dpart_hbm,
             src_v, dst_v, s_v, tacc_v, dacc_v, buf_v, red_v, tsh, dsh, sem):
    cid = lax.axis_index("c")
    sid = lax.axis_index("s")
    wid = sid * NC + cid
    base = wid * E_PER_W
    c1 = pltpu.async_copy(ei_hbm.at[0, pl.ds(base, E_PER_W)], src_v, sem)
    c2 = pltpu.async_copy(ei_hbm.at[1, pl.ds(base, E_PER_W)], dst_v, sem)
    c3 = pltpu.async_copy(s_hbm.at[0], s_v, sem)
    c1.wait()
    c2.wait()
    c3.wait()

    zeros = jnp.zeros((L,), jnp.float32)

    @plsc.parallel_loop(0, N_PAD // L, unroll=8)
    def _(i):
        tacc_v[pl.ds(i * L, L)] = zeros
        dacc_v[pl.ds(i * L, L)] = zeros

    ones = jnp.ones((L,), jnp.float32)

    @plsc.parallel_loop(0, E_PER_W // L, unroll=16)
    def _(i):
        off = i * L
        i_s = src_v[pl.ds(off, L)]
        i_d = dst_v[pl.ds(off, L)]
        vals = plsc.load_gather(s_v, [i_s])
        plsc.addupdate_scatter(tacc_v, [i_d], vals)
        plsc.addupdate_scatter(dacc_v, [i_s], ones)

    p1 = pltpu.async_copy(tacc_v, tsh.at[sid], sem)
    p2 = pltpu.async_copy(dacc_v, dsh.at[sid], sem)
    p1.wait()
    p2.wait()
    plsc.subcore_barrier()

    c0 = sid * COLS2
    for arr_sh, out_hbm in ((tsh, tpart_hbm), (dsh, dpart_hbm)):
        cps = [pltpu.async_copy(arr_sh.at[r, pl.ds(c0, COLS2)], buf_v.at[r], sem)
               for r in range(NS)]
        for cp in cps:
            cp.wait()

        @plsc.parallel_loop(0, COLS2 // L, unroll=4)
        def _(j):
            acc = buf_v[0, pl.ds(j * L, L)]
            for r in range(1, NS):
                acc = acc + buf_v[r, pl.ds(j * L, L)]
            red_v[pl.ds(j * L, L)] = acc

        pltpu.sync_copy(red_v, out_hbm.at[cid, pl.ds(c0, COLS2)])


def _k2(ei, s):
    mesh = plsc.VectorSubcoreMesh(core_axis_name="c", subcore_axis_name="s")
    f = pl.kernel(
        _k2_body,
        mesh=mesh,
        compiler_params=_SC_PARAMS,
        out_type=(
            jax.ShapeDtypeStruct((NC, N_PAD), jnp.float32),
            jax.ShapeDtypeStruct((NC, N_PAD), jnp.float32),
        ),
        scratch_types=[
            pltpu.VMEM((E_PER_W,), jnp.int32),
            pltpu.VMEM((E_PER_W,), jnp.int32),
            pltpu.VMEM((N_PAD,), jnp.float32),
            pltpu.VMEM((N_PAD,), jnp.float32),
            pltpu.VMEM((N_PAD,), jnp.float32),
            pltpu.VMEM((NS, COLS2), jnp.float32),
            pltpu.VMEM((COLS2,), jnp.float32),
            pltpu.VMEM_SHARED((NS, N_PAD), jnp.float32),
            pltpu.VMEM_SHARED((NS, N_PAD), jnp.float32),
            pltpu.SemaphoreType.DMA,
        ],
    )
    return f(ei, s)


def _k4_body(ei_hbm, t_hbm, deg_hbm, out_hbm,
             src_v, dst_v, t_v, deg_v, tmp_v, tmp2_v, res_v, sem):
    wid = lax.axis_index("s") * NC + lax.axis_index("c")
    base = wid * E_PER_W
    cps = [
        pltpu.async_copy(ei_hbm.at[0, pl.ds(base, E_PER_W)], src_v, sem),
        pltpu.async_copy(ei_hbm.at[1, pl.ds(base, E_PER_W)], dst_v, sem),
        pltpu.async_copy(t_hbm.at[0, pl.ds(0, N_NODES)], t_v, sem),
        pltpu.async_copy(t_hbm.at[1, pl.ds(0, N_NODES)], tmp_v, sem),
        pltpu.async_copy(deg_hbm.at[0, pl.ds(0, N_NODES)], deg_v, sem),
        pltpu.async_copy(deg_hbm.at[1, pl.ds(0, N_NODES)], tmp2_v, sem),
    ]
    for cp in cps:
        cp.wait()

    @plsc.parallel_loop(0, N_NODES // L, unroll=8)
    def _(i):
        sl = pl.ds(i * L, L)
        t_v[sl] = t_v[sl] + tmp_v[sl]
        deg_v[sl] = deg_v[sl] + tmp2_v[sl]

    @plsc.parallel_loop(0, E_PER_W // L, unroll=16,
                        carry=jnp.zeros((L,), jnp.float32))
    def acc(i, a):
        off = i * L
        i_s = src_v[pl.ds(off, L)]
        i_d = dst_v[pl.ds(off, L)]
        tv = plsc.load_gather(t_v, [i_s])
        dv = plsc.load_gather(deg_v, [i_d])
        return a + tv * dv

    res_v[...] = acc
    pltpu.sync_copy(res_v, out_hbm.at[wid])


def _k4(ei, t2, deg2):
    mesh = plsc.VectorSubcoreMesh(core_axis_name="c", subcore_axis_name="s")
    f = pl.kernel(
        _k4_body,
        mesh=mesh,
        compiler_params=_SC_PARAMS,
        out_type=jax.ShapeDtypeStruct((NW, L), jnp.float32),
        scratch_types=[
            pltpu.VMEM((E_PER_W,), jnp.int32),
            pltpu.VMEM((E_PER_W,), jnp.int32),
            pltpu.VMEM((N_NODES,), jnp.float32),
            pltpu.VMEM((N_NODES,), jnp.float32),
            pltpu.VMEM((N_NODES,), jnp.float32),
            pltpu.VMEM((N_NODES,), jnp.float32),
            pltpu.VMEM((L,), jnp.float32),
            pltpu.SemaphoreType.DMA,
        ],
    )
    return f(ei, t2, deg2)


def _k5_body(parts_ref, b_ref, out_ref):
    total = jnp.sum(parts_ref[...]) * (1.0 / N_NODES) + b_ref[0, 0]
    out_ref[...] = total.reshape(1, 1)


def _k5(parts, b_read):
    return pl.pallas_call(
        _k5_body,
        out_shape=jax.ShapeDtypeStruct((1, 1), jnp.float32),
    )(parts, b_read.reshape(1, 1))


def kernel(x, edge_index, W_pool, W_read, b_read):
    ei = edge_index.astype(jnp.int32)
    s = _k1(x, W_pool, W_read)
    tpart, dpart = _k2(ei, s)
    parts = _k4(ei, tpart, dpart)
    out = _k5(parts, b_read)
    return out.reshape(1)
